# baseline (speedup 1.0000x reference)
import functools

import jax
import jax.numpy as jnp
from jax import lax
from jax.experimental import pallas as pl
from jax.experimental.pallas import tpu as pltpu
from jax.experimental.pallas import tpu_sc as plsc

E = 8
K = 2
D = 1024
DFF = 4096
T = 2048
CAP = 640
NE = T * K
XROWS = E * CAP + 8
TRASH = E * CAP
CHUNK = 256
NCHUNK = NE // CHUNK
DFF_BLK = 2048
NDFF = DFF // DFF_BLK

NW = 32
TPW = T // NW


def _router_body(x_ref, wr_ref, locd_ref, locc_ref, wv_ref, cnt_ref, h_ref, wv_scr):
    xv = x_ref[...]
    wr = wr_ref[...]
    logits = lax.dot_general(
        xv, wr, (((1,), (1,)), ((), ())),
        preferred_element_type=jnp.float32,
    )
    logits = jnp.clip(logits, -100.0, 100.0)

    iota_e = lax.broadcasted_iota(jnp.int32, (T, E), 1)
    l1 = jnp.max(logits, axis=1, keepdims=True)
    i1 = jnp.min(jnp.where(logits == l1, iota_e, E), axis=1, keepdims=True)
    masked = jnp.where(iota_e == i1, -3e38, logits)
    l2 = jnp.max(masked, axis=1, keepdims=True)
    i2 = jnp.min(jnp.where(masked == l2, iota_e, E), axis=1, keepdims=True)

    w0 = 1.0 / (1.0 + jnp.exp(l2 - l1))
    w1 = 1.0 - w0

    h_ref[pl.ds(0, T), :] = (iota_e == i1).astype(jnp.float32)
    h_ref[pl.ds(T, T), :] = (iota_e == i2).astype(jnp.float32)
    wv_scr[pl.ds(0, T), :] = w0
    wv_scr[pl.ds(T, T), :] = w1

    tri = (
        lax.broadcasted_iota(jnp.int32, (CHUNK, CHUNK), 1)
        < lax.broadcasted_iota(jnp.int32, (CHUNK, CHUNK), 0)
    ).astype(jnp.float32)
    iota_ef = lax.broadcasted_iota(jnp.int32, (CHUNK, E), 1).astype(jnp.float32)

    def body(c, carry):
        hc = h_ref[pl.ds(c * CHUNK, CHUNK), :]
        counts = lax.dot_general(
            tri, hc, (((1,), (0,)), ((), ())),
            preferred_element_type=jnp.float32,
        ) + carry
        slot = jnp.sum(counts * hc, axis=1, keepdims=True)
        ef = jnp.sum(iota_ef * hc, axis=1, keepdims=True)
        valid = slot < float(CAP)
        base = ef * float(CAP) + slot
        locd = jnp.where(valid, base, float(TRASH))
        locc = jnp.where(valid, base, ef * float(CAP))
        wch = wv_scr[pl.ds(c * CHUNK, CHUNK), :]
        locd_ref[pl.ds(c * CHUNK, CHUNK), :] = locd.astype(jnp.int32)
        locc_ref[pl.ds(c * CHUNK, CHUNK), :] = locc.astype(jnp.int32)
        wv_ref[pl.ds(c * CHUNK, CHUNK), :] = jnp.where(valid, wch, 0.0)
        return carry + jnp.sum(hc, axis=0, keepdims=True)

    totals = lax.fori_loop(0, NCHUNK, body, jnp.zeros((1, E), jnp.float32))
    cnt_ref[...] = jnp.minimum(totals, float(CAP)).astype(jnp.int32)


def _router(xf, w_router):
    return pl.pallas_call(
        _router_body,
        out_shape=(
            jax.ShapeDtypeStruct((NE, 1), jnp.int32),
            jax.ShapeDtypeStruct((NE, 1), jnp.int32),
            jax.ShapeDtypeStruct((NE, 1), jnp.float32),
            jax.ShapeDtypeStruct((1, E), jnp.int32),
        ),
        scratch_shapes=[
            pltpu.VMEM((NE, E), jnp.float32),
            pltpu.VMEM((NE, 1), jnp.float32),
        ],
    )(xf, w_router)


@functools.lru_cache(maxsize=None)
def _sc_kernels():
    mesh = plsc.VectorSubcoreMesh(core_axis_name="c", subcore_axis_name="s")

    @functools.partial(
        pl.kernel,
        mesh=mesh,
        out_type=jax.ShapeDtypeStruct((XROWS, D), jnp.float32),
        scratch_types=[
            pltpu.VMEM((TPW, D), jnp.float32),
            pltpu.VMEM((TPW,), jnp.int32),
            pltpu.VMEM((TPW,), jnp.int32),
            pltpu.SemaphoreType.DMA,
        ],
    )
    def _dispatch(x_hbm, loc_hbm, xbuf_hbm, rows_v, idx0_v, idx1_v, sem):
        wid = lax.axis_index("s") * 2 + lax.axis_index("c")
        base = wid * TPW
        pltpu.sync_copy(x_hbm.at[pl.ds(base, TPW)], rows_v)
        pltpu.sync_copy(loc_hbm.at[wid, 0], idx0_v)
        pltpu.sync_copy(loc_hbm.at[wid, 1], idx1_v)
        pltpu.async_copy(rows_v, xbuf_hbm.at[idx0_v], sem).wait()
        pltpu.async_copy(rows_v, xbuf_hbm.at[idx1_v], sem).wait()

    @functools.partial(
        pl.kernel,
        mesh=mesh,
        out_type=(
            jax.ShapeDtypeStruct((T, D), jnp.float32),
            jax.ShapeDtypeStruct((T, D), jnp.float32),
        ),
        scratch_types=[
            pltpu.VMEM((TPW, D), jnp.float32),
            pltpu.VMEM((TPW,), jnp.int32),
            pltpu.SemaphoreType.DMA,
        ],
    )
    def _combine_sc(o_hbm, loc_hbm, out0_hbm, out1_hbm, rows_v, idx_v, sem):
        wid = lax.axis_index("s") * 2 + lax.axis_index("c")
        base = wid * TPW
        pltpu.sync_copy(loc_hbm.at[wid, 0], idx_v)
        pltpu.async_copy(o_hbm.at[idx_v], rows_v, sem).wait()
        pltpu.sync_copy(rows_v, out0_hbm.at[pl.ds(base, TPW)])
        pltpu.sync_copy(loc_hbm.at[wid, 1], idx_v)
        pltpu.async_copy(o_hbm.at[idx_v], rows_v, sem).wait()
        pltpu.sync_copy(rows_v, out1_hbm.at[pl.ds(base, TPW)])

    return _dispatch, _combine_sc


def _ffn_body(x_ref, wu_ref, bu_ref, wd_ref, bd_ref, o_ref):
    d = pl.program_id(1)
    xv = x_ref[...].astype(jnp.bfloat16)
    wu = wu_ref[0]
    h = lax.dot_general(
        xv, wu, (((1,), (1,)), ((), ())), preferred_element_type=jnp.float32
    ) + bu_ref[0]
    h = 0.5 * h * (1.0 + lax.erf(h * 0.7071067811865476))
    h = jnp.clip(h, -1000.0, 1000.0)
    contrib = lax.dot_general(
        h.astype(jnp.bfloat16), wd_ref[0], (((1,), (1,)), ((), ())),
        preferred_element_type=jnp.float32,
    )

    @pl.when(d == 0)
    def _():
        o_ref[...] = jnp.broadcast_to(bd_ref[0], (CAP, D))

    o_ref[...] += contrib

    @pl.when(d == NDFF - 1)
    def _():
        o_ref[...] = jnp.clip(o_ref[...], -1000.0, 1000.0)


def _ffn(counts, xbuf, w_up, b_up, w_down, b_down):
    del counts
    return pl.pallas_call(
        _ffn_body,
        grid=(E, NDFF),
        in_specs=[
            pl.BlockSpec((CAP, D), lambda e, d: (e, 0)),
            pl.BlockSpec((1, DFF_BLK, D), lambda e, d: (e, d, 0)),
            pl.BlockSpec((1, 1, DFF_BLK), lambda e, d: (e, 0, d)),
            pl.BlockSpec((1, D, DFF_BLK), lambda e, d: (e, 0, d)),
            pl.BlockSpec((1, 1, D), lambda e, d: (e, 0, 0)),
        ],
        out_specs=pl.BlockSpec((CAP, D), lambda e, d: (e, 0)),
        out_shape=jax.ShapeDtypeStruct((E * CAP, D), jnp.float32),
    )(
        xbuf,
        w_up.astype(jnp.bfloat16),
        b_up.reshape(E, 1, DFF),
        w_down.astype(jnp.bfloat16),
        b_down.reshape(E, 1, D),
    )


def _combine_body(o0_ref, o1_ref, w0_ref, w1_ref, out_ref):
    out_ref[...] = w0_ref[...] * o0_ref[...] + w1_ref[...] * o1_ref[...]


def _combine_tc(o0, o1, w0, w1):
    blk = T // 8
    return pl.pallas_call(
        _combine_body,
        grid=(8,),
        in_specs=[
            pl.BlockSpec((blk, D), lambda m: (m, 0)),
            pl.BlockSpec((blk, D), lambda m: (m, 0)),
            pl.BlockSpec((blk, 1), lambda m: (m, 0)),
            pl.BlockSpec((blk, 1), lambda m: (m, 0)),
        ],
        out_specs=pl.BlockSpec((blk, D), lambda m: (m, 0)),
        out_shape=jax.ShapeDtypeStruct((T, D), jnp.float32),
    )(o0, o1, w0, w1)


def kernel(x, W_router, W_up, b_up, W_down, b_down):
    Bb, Ss, Dm = x.shape
    xf = x.reshape(Bb * Ss, Dm)
    locd, locc, wv, counts = _router(xf, W_router)
    locd3 = locd.reshape(K, NW, TPW).transpose(1, 0, 2)
    locc3 = locc.reshape(K, NW, TPW).transpose(1, 0, 2)
    w0 = wv[:T].reshape(T, 1)
    w1 = wv[T:].reshape(T, 1)
    dispatch, combine_sc = _sc_kernels()
    xbuf = dispatch(xf, locd3)
    obuf = _ffn(counts.reshape(E), xbuf, W_up, b_up, W_down, b_down)
    o0, o1 = combine_sc(obuf, locc3)
    out = _combine_tc(o0, o1, w0, w1)
    return out.reshape(Bb, Ss, Dm)


# speedup vs baseline: 1.6847x; 1.6847x over previous
import functools

import jax
import jax.numpy as jnp
from jax import lax
from jax.experimental import pallas as pl
from jax.experimental.pallas import tpu as pltpu
from jax.experimental.pallas import tpu_sc as plsc

E = 8
K = 2
D = 1024
DFF = 4096
T = 2048
CAP = 640
NE = T * K
XROWS = E * CAP + 8
TRASH = E * CAP
CHUNK = 256
NCHUNK = NE // CHUNK
DFF_BLK = 2048
NDFF = DFF // DFF_BLK

NW = 32
TPW = T // NW


def _router_body(x_ref, wr_ref, locd_ref, locc_ref, wv_ref, cnt_ref, h_ref, wv_scr):
    xv = x_ref[...]
    wr = wr_ref[...]
    logits = lax.dot_general(
        xv, wr, (((1,), (1,)), ((), ())),
        preferred_element_type=jnp.float32,
    )
    logits = jnp.clip(logits, -100.0, 100.0)

    iota_e = lax.broadcasted_iota(jnp.int32, (T, E), 1)
    l1 = jnp.max(logits, axis=1, keepdims=True)
    i1 = jnp.min(jnp.where(logits == l1, iota_e, E), axis=1, keepdims=True)
    masked = jnp.where(iota_e == i1, -3e38, logits)
    l2 = jnp.max(masked, axis=1, keepdims=True)
    i2 = jnp.min(jnp.where(masked == l2, iota_e, E), axis=1, keepdims=True)

    w0 = 1.0 / (1.0 + jnp.exp(l2 - l1))
    w1 = 1.0 - w0

    h_ref[pl.ds(0, T), :] = (iota_e == i1).astype(jnp.float32)
    h_ref[pl.ds(T, T), :] = (iota_e == i2).astype(jnp.float32)
    wv_scr[pl.ds(0, T), :] = w0
    wv_scr[pl.ds(T, T), :] = w1

    tri = (
        lax.broadcasted_iota(jnp.int32, (CHUNK, CHUNK), 1)
        < lax.broadcasted_iota(jnp.int32, (CHUNK, CHUNK), 0)
    ).astype(jnp.float32)
    iota_ef = lax.broadcasted_iota(jnp.int32, (CHUNK, E), 1).astype(jnp.float32)

    def body(c, carry):
        hc = h_ref[pl.ds(c * CHUNK, CHUNK), :]
        counts = lax.dot_general(
            tri, hc, (((1,), (0,)), ((), ())),
            preferred_element_type=jnp.float32,
        ) + carry
        slot = jnp.sum(counts * hc, axis=1, keepdims=True)
        ef = jnp.sum(iota_ef * hc, axis=1, keepdims=True)
        valid = slot < float(CAP)
        base = ef * float(CAP) + slot
        locd = jnp.where(valid, base, float(TRASH))
        locc = jnp.where(valid, base, ef * float(CAP))
        wch = wv_scr[pl.ds(c * CHUNK, CHUNK), :]
        locd_ref[pl.ds(c * CHUNK, CHUNK), :] = locd.astype(jnp.int32)
        locc_ref[pl.ds(c * CHUNK, CHUNK), :] = locc.astype(jnp.int32)
        wv_ref[pl.ds(c * CHUNK, CHUNK), :] = jnp.where(valid, wch, 0.0)
        return carry + jnp.sum(hc, axis=0, keepdims=True)

    totals = lax.fori_loop(0, NCHUNK, body, jnp.zeros((1, E), jnp.float32))
    cnt_ref[...] = jnp.minimum(totals, float(CAP)).astype(jnp.int32)


def _router(xf, w_router):
    return pl.pallas_call(
        _router_body,
        out_shape=(
            jax.ShapeDtypeStruct((NE, 1), jnp.int32),
            jax.ShapeDtypeStruct((NE, 1), jnp.int32),
            jax.ShapeDtypeStruct((NE, 1), jnp.float32),
            jax.ShapeDtypeStruct((1, E), jnp.int32),
        ),
        scratch_shapes=[
            pltpu.VMEM((NE, E), jnp.float32),
            pltpu.VMEM((NE, 1), jnp.float32),
        ],
    )(xf, w_router)


@functools.lru_cache(maxsize=None)
def _sc_kernels():
    mesh = plsc.VectorSubcoreMesh(core_axis_name="c", subcore_axis_name="s")

    @functools.partial(
        pl.kernel,
        mesh=mesh,
        out_type=jax.ShapeDtypeStruct((XROWS, D), jnp.float32),
        scratch_types=[
            pltpu.VMEM((TPW, D), jnp.float32),
            pltpu.VMEM((TPW,), jnp.int32),
            pltpu.VMEM((TPW,), jnp.int32),
            pltpu.SemaphoreType.DMA,
        ],
    )
    def _dispatch(x_hbm, loc_hbm, xbuf_hbm, rows_v, idx0_v, idx1_v, sem):
        wid = lax.axis_index("s") * 2 + lax.axis_index("c")
        base = wid * TPW
        pltpu.sync_copy(x_hbm.at[pl.ds(base, TPW)], rows_v)
        pltpu.sync_copy(loc_hbm.at[wid, 0], idx0_v)
        pltpu.sync_copy(loc_hbm.at[wid, 1], idx1_v)
        pltpu.async_copy(rows_v, xbuf_hbm.at[idx0_v], sem).wait()
        pltpu.async_copy(rows_v, xbuf_hbm.at[idx1_v], sem).wait()

    HALF = TPW // 2

    @functools.partial(
        pl.kernel,
        mesh=mesh,
        out_type=jax.ShapeDtypeStruct((T, D), jnp.float32),
        scratch_types=[
            pltpu.VMEM((HALF, D), jnp.float32),
            pltpu.VMEM((HALF, D), jnp.float32),
            pltpu.VMEM((TPW,), jnp.int32),
            pltpu.VMEM((TPW,), jnp.int32),
            pltpu.VMEM((HALF, 16), jnp.float32),
            pltpu.VMEM((HALF, 16), jnp.float32),
            pltpu.SemaphoreType.DMA,
            pltpu.SemaphoreType.DMA,
        ],
    )
    def _combine_sc(o_hbm, loc_hbm, wb_hbm, out_hbm,
                    r0_v, r1_v, idx0_v, idx1_v, w0_v, w1_v, sem0, sem1):
        wid = lax.axis_index("s") * 2 + lax.axis_index("c")
        base = wid * TPW
        pltpu.sync_copy(loc_hbm.at[wid, 0], idx0_v)
        pltpu.sync_copy(loc_hbm.at[wid, 1], idx1_v)
        for half in range(2):
            hbase = base + half * HALF
            g0 = pltpu.async_copy(
                o_hbm.at[idx0_v.at[pl.ds(half * HALF, HALF)]], r0_v, sem0)
            g1 = pltpu.async_copy(
                o_hbm.at[idx1_v.at[pl.ds(half * HALF, HALF)]], r1_v, sem1)
            pltpu.sync_copy(wb_hbm.at[pl.ds(hbase, HALF)], w0_v)
            pltpu.sync_copy(wb_hbm.at[pl.ds(T + hbase, HALF)], w1_v)
            g0.wait()
            g1.wait()

            def row_body(r, _):
                wv0 = w0_v[r]
                wv1 = w1_v[r]
                for v in range(D // 16):
                    sl = pl.ds(v * 16, 16)
                    r0_v[r, sl] = r0_v[r, sl] * wv0 + r1_v[r, sl] * wv1
                return 0

            lax.fori_loop(0, HALF, row_body, 0)
            pltpu.sync_copy(r0_v, out_hbm.at[pl.ds(hbase, HALF)])

    return _dispatch, _combine_sc


def _ffn_body(x_ref, wu_ref, bu_ref, wd_ref, bd_ref, o_ref):
    d = pl.program_id(1)
    xv = x_ref[...]
    wu = wu_ref[0]
    h = lax.dot_general(
        xv, wu, (((1,), (1,)), ((), ())), preferred_element_type=jnp.float32
    ) + bu_ref[0]
    h = 0.5 * h * (1.0 + lax.erf(h * 0.7071067811865476))
    h = jnp.clip(h, -1000.0, 1000.0)
    contrib = lax.dot_general(
        h, wd_ref[0], (((1,), (1,)), ((), ())), preferred_element_type=jnp.float32
    )

    @pl.when(d == 0)
    def _():
        o_ref[...] = jnp.broadcast_to(bd_ref[0], (CAP, D))

    o_ref[...] += contrib

    @pl.when(d == NDFF - 1)
    def _():
        o_ref[...] = jnp.clip(o_ref[...], -1000.0, 1000.0)


def _ffn(counts, xbuf, w_up, b_up, w_down, b_down):
    del counts
    return pl.pallas_call(
        _ffn_body,
        grid=(E, NDFF),
        in_specs=[
            pl.BlockSpec((CAP, D), lambda e, d: (e, 0)),
            pl.BlockSpec((1, DFF_BLK, D), lambda e, d: (e, d, 0)),
            pl.BlockSpec((1, 1, DFF_BLK), lambda e, d: (e, 0, d)),
            pl.BlockSpec((1, D, DFF_BLK), lambda e, d: (e, 0, d)),
            pl.BlockSpec((1, 1, D), lambda e, d: (e, 0, 0)),
        ],
        out_specs=pl.BlockSpec((CAP, D), lambda e, d: (e, 0)),
        out_shape=jax.ShapeDtypeStruct((E * CAP, D), jnp.float32),
    )(xbuf, w_up, b_up.reshape(E, 1, DFF), w_down, b_down.reshape(E, 1, D))


def kernel(x, W_router, W_up, b_up, W_down, b_down):
    Bb, Ss, Dm = x.shape
    xf = x.reshape(Bb * Ss, Dm)
    locd, locc, wv, counts = _router(xf, W_router)
    locd3 = locd.reshape(K, NW, TPW).transpose(1, 0, 2)
    locc3 = locc.reshape(K, NW, TPW).transpose(1, 0, 2)
    wvb = jnp.broadcast_to(wv, (NE, 16))
    dispatch, combine_sc = _sc_kernels()
    xbuf = dispatch(xf, locd3)
    obuf = _ffn(counts.reshape(E), xbuf, W_up, b_up, W_down, b_down)
    out = combine_sc(obuf, locc3, wvb)
    return out.reshape(Bb, Ss, Dm)
